# baseline (speedup 1.0000x reference)
import functools

import jax
import jax.numpy as jnp
from jax import lax
from jax.experimental import pallas as pl
from jax.experimental.pallas import tpu as pltpu
from jax.experimental.pallas import tpu_sc as plsc

DIM = 32
B = 16384
NC = 2
NS = 16
NW = NC * NS
BPW = B // NW

_sc_mesh = plsc.VectorSubcoreMesh(core_axis_name="c", subcore_axis_name="s")


@functools.partial(
    pl.kernel,
    out_type=[jax.ShapeDtypeStruct((B, DIM), jnp.float32)] * 4,
    mesh=_sc_mesh,
    scratch_types=[
        pltpu.VMEM((BPW,), jnp.int32),
        pltpu.SemaphoreType.DMA,
    ],
)
def _sc_gather4(idx_all, ut, it, st, gt, ue_o, ie_o, se_o, ge_o,
                idx_v, sem):
    wid = lax.axis_index("s") * NC + lax.axis_index("c")
    base = wid * BPW
    outs = (ue_o, ie_o, se_o, ge_o)
    for t, tab in enumerate((ut, it, st, gt)):
        pltpu.sync_copy(idx_all.at[t, wid], idx_v)
        o = outs[t]

        def body(g, _, tab=tab, o=o):
            vec = idx_v[pl.ds(g * 16, 16)]
            for j in range(16):
                pltpu.async_copy(tab.at[pl.ds(vec[j], 1)],
                                 o.at[pl.ds(base + g * 16 + j, 1)], sem)
            return 0

        lax.fori_loop(0, BPW // 16, body, 0)
    for o in outs:
        pltpu.make_async_copy(o.at[pl.ds(0, BPW)],
                              o.at[pl.ds(base, BPW)], sem).wait()


TB = 2048


def _mlp_body(ue, ie, se, ge, w1u, w1i, w1s, w1g, b1, w2, b2, w3, b3, w4, b4,
              wo, bo, out):
    h = (jnp.dot(ue[...], w1u[...], preferred_element_type=jnp.float32)
         + jnp.dot(ie[...], w1i[...], preferred_element_type=jnp.float32)
         + jnp.dot(se[...], w1s[...], preferred_element_type=jnp.float32)
         + jnp.dot(ge[...], w1g[...], preferred_element_type=jnp.float32)
         + b1[...])
    h = jnp.maximum(h, 0.0)
    h = jnp.maximum(jnp.dot(h, w2[...], preferred_element_type=jnp.float32)
                    + b2[...], 0.0)
    h = jnp.maximum(jnp.dot(h, w3[...], preferred_element_type=jnp.float32)
                    + b3[...], 0.0)
    h = jnp.maximum(jnp.dot(h, w4[...], preferred_element_type=jnp.float32)
                    + b4[...], 0.0)
    out[...] = jnp.dot(h, wo[...], preferred_element_type=jnp.float32) + bo[...]


def _mlp(ue, ie, se, ge, W1, b1, W2, b2, W3, b3, W4, b4, Wo, bo):
    w1t = W1.T
    full = lambda shape: pl.BlockSpec(shape, lambda i: (0, 0))
    emb = pl.BlockSpec((TB, DIM), lambda i: (i, 0))
    return pl.pallas_call(
        _mlp_body,
        grid=(B // TB,),
        in_specs=[emb, emb, emb, emb,
                  full((DIM, 64)), full((DIM, 64)), full((DIM, 64)),
                  full((DIM, 64)), full((1, 64)),
                  full((64, 32)), full((1, 32)),
                  full((32, 16)), full((1, 16)),
                  full((16, 8)), full((1, 8)),
                  full((8, 1)), full((1, 1))],
        out_specs=pl.BlockSpec((TB, 1), lambda i: (i, 0)),
        out_shape=jax.ShapeDtypeStruct((B, 1), jnp.float32),
        compiler_params=pltpu.CompilerParams(
            dimension_semantics=("arbitrary",)),
    )(ue, ie, se, ge,
      w1t[0:DIM], w1t[DIM:2 * DIM], w1t[2 * DIM:3 * DIM], w1t[3 * DIM:],
      b1.reshape(1, 64), W2.T, b2.reshape(1, 32), W3.T, b3.reshape(1, 16),
      W4.T, b4.reshape(1, 8), Wo.T, bo.reshape(1, 1))


def kernel(user_indices, item_indices, social_indices, giver_indices,
           user_table, item_table, social_table, giver_table,
           W1, b1, W2, b2, W3, b3, W4, b4, Wo, bo):
    nrow = user_table.shape[0] - 1
    idx_all = jnp.stack([
        jnp.clip(user_indices, 0, nrow),
        jnp.clip(item_indices, 0, nrow),
        jnp.clip(social_indices, 0, nrow),
        jnp.clip(giver_indices, 0, nrow),
    ]).reshape(4, NW, BPW)
    ue, ie, se, ge = _sc_gather4(idx_all, user_table, item_table,
                                 social_table, giver_table)
    pred = _mlp(ue, ie, se, ge, W1, b1, W2, b2, W3, b3, W4, b4, Wo, bo)
    return pred.reshape(-1)


# speedup vs baseline: 1.2438x; 1.2438x over previous
import functools

import jax
import jax.numpy as jnp
from jax import lax
from jax.experimental import pallas as pl
from jax.experimental.pallas import tpu as pltpu
from jax.experimental.pallas import tpu_sc as plsc

DIM = 32
LINE = 128
B = 16384
NC = 2
NS = 16
NW = NC * NS
BPW = B // NW
CHUNK = 128
NCHUNK = BPW // CHUNK

_sc_mesh = plsc.VectorSubcoreMesh(core_axis_name="c", subcore_axis_name="s")


@functools.partial(
    pl.kernel,
    out_type=[jax.ShapeDtypeStruct((B, LINE), jnp.float32)] * 4,
    mesh=_sc_mesh,
    scratch_types=[
        pltpu.VMEM((NCHUNK, CHUNK), jnp.int32),
        pltpu.VMEM((NCHUNK, CHUNK, LINE), jnp.float32),
        pltpu.SemaphoreType.DMA,
    ],
)
def _sc_gather4(line_all, ut, it, st, gt, ue_o, ie_o, se_o, ge_o,
                idx_v, rows_v, sem):
    wid = lax.axis_index("s") * NC + lax.axis_index("c")
    base = wid * BPW
    outs = (ue_o, ie_o, se_o, ge_o)
    for t, tab in enumerate((ut, it, st, gt)):
        pltpu.sync_copy(line_all.at[t, wid], idx_v)
        cps = [pltpu.async_copy(tab.at[idx_v.at[j]], rows_v.at[j], sem)
               for j in range(NCHUNK)]
        for cp in cps:
            cp.wait()
        for j in range(NCHUNK):
            pltpu.sync_copy(rows_v.at[j],
                            outs[t].at[pl.ds(base + j * CHUNK, CHUNK)])


TB = 2048


def _emb_select(lines, sel, last):
    e = jnp.where(sel == 0, lines[:, 0 * DIM:1 * DIM],
        jnp.where(sel == 1, lines[:, 1 * DIM:2 * DIM],
        jnp.where(sel == 2, lines[:, 2 * DIM:3 * DIM],
                  lines[:, 3 * DIM:4 * DIM])))
    return jnp.where(sel < 0, last, e)


def _mlp_body(ul, il, sl, gl, us, is_, ss, gs, lu, li, ls, lg,
              w1u, w1i, w1s, w1g, b1, w2, b2, w3, b3, w4, b4, wo, bo, out):
    eu = _emb_select(ul[...], us[...], lu[...])
    ei = _emb_select(il[...], is_[...], li[...])
    es = _emb_select(sl[...], ss[...], ls[...])
    eg = _emb_select(gl[...], gs[...], lg[...])
    h = (jnp.dot(eu, w1u[...], preferred_element_type=jnp.float32)
         + jnp.dot(ei, w1i[...], preferred_element_type=jnp.float32)
         + jnp.dot(es, w1s[...], preferred_element_type=jnp.float32)
         + jnp.dot(eg, w1g[...], preferred_element_type=jnp.float32)
         + b1[...])
    h = jnp.maximum(h, 0.0)
    h = jnp.maximum(jnp.dot(h, w2[...], preferred_element_type=jnp.float32)
                    + b2[...], 0.0)
    h = jnp.maximum(jnp.dot(h, w3[...], preferred_element_type=jnp.float32)
                    + b3[...], 0.0)
    h = jnp.maximum(jnp.dot(h, w4[...], preferred_element_type=jnp.float32)
                    + b4[...], 0.0)
    out[...] = jnp.dot(h, wo[...], preferred_element_type=jnp.float32) + bo[...]


def _mlp(lines4, sel4, last4, W1, b1, W2, b2, W3, b3, W4, b4, Wo, bo):
    w1t = W1.T
    full = lambda shape: pl.BlockSpec(shape, lambda i: (0, 0))
    linesb = pl.BlockSpec((TB, LINE), lambda i: (i, 0))
    selb = pl.BlockSpec((TB, 1), lambda i: (i, 0))
    return pl.pallas_call(
        _mlp_body,
        grid=(B // TB,),
        in_specs=[linesb] * 4 + [selb] * 4
                 + [full((1, DIM))] * 4
                 + [full((DIM, 64)), full((DIM, 64)), full((DIM, 64)),
                    full((DIM, 64)), full((1, 64)),
                    full((64, 32)), full((1, 32)),
                    full((32, 16)), full((1, 16)),
                    full((16, 8)), full((1, 8)),
                    full((8, 1)), full((1, 1))],
        out_specs=pl.BlockSpec((TB, 1), lambda i: (i, 0)),
        out_shape=jax.ShapeDtypeStruct((B, 1), jnp.float32),
        compiler_params=pltpu.CompilerParams(
            dimension_semantics=("arbitrary",)),
    )(*lines4, *sel4, *last4,
      w1t[0:DIM], w1t[DIM:2 * DIM], w1t[2 * DIM:3 * DIM], w1t[3 * DIM:],
      b1.reshape(1, 64), W2.T, b2.reshape(1, 32), W3.T, b3.reshape(1, 16),
      W4.T, b4.reshape(1, 8), Wo.T, bo.reshape(1, 1))


def kernel(user_indices, item_indices, social_indices, giver_indices,
           user_table, item_table, social_table, giver_table,
           W1, b1, W2, b2, W3, b3, W4, b4, Wo, bo):
    ntot = user_table.shape[0]
    nline = (ntot - 1) // 4
    tables = (user_table, item_table, social_table, giver_table)
    views = [t[:nline * 4].reshape(nline, LINE) for t in tables]
    lasts = [t[ntot - 1].reshape(1, DIM) for t in tables]
    idxs = [jnp.clip(x, 0, ntot - 1)
            for x in (user_indices, item_indices, social_indices,
                      giver_indices)]
    line_all = jnp.stack(
        [jnp.minimum(x, nline * 4 - 1) // 4 for x in idxs]
    ).reshape(4, NW, NCHUNK, CHUNK)
    sel4 = [jnp.where(x >= nline * 4, -1, x % 4).astype(jnp.int32)
            .reshape(B, 1) for x in idxs]
    lines4 = _sc_gather4(line_all, *views)
    pred = _mlp(lines4, sel4, lasts, W1, b1, W2, b2, W3, b3, W4, b4, Wo, bo)
    return pred.reshape(-1)
